# Initial kernel scaffold; baseline (speedup 1.0000x reference)
import jax
import jax.numpy as jnp
from jax.experimental import pallas as pl


def kernel(input_ids, embed_weight):
    raise NotImplementedError("write your pallas kernel here")


# speedup vs baseline: 1.4328x; 1.4328x over previous
import functools

import jax
import jax.numpy as jnp
from jax import lax
from jax.experimental import pallas as pl
from jax.experimental.pallas import tpu as pltpu
from jax.experimental.pallas import tpu_sc as plsc

EMBED_DIM = 384
CHUNK = 128


@functools.lru_cache(maxsize=None)
def _make_lookup(B, D):
    info = plsc.get_sparse_core_info()
    NC, NS = info.num_cores, info.num_subcores
    NW = NC * NS
    assert B % (NW * CHUNK) == 0
    b_per_w = B // NW
    n_chunks = b_per_w // CHUNK

    mesh = plsc.VectorSubcoreMesh(core_axis_name="c", subcore_axis_name="s")

    @functools.partial(
        pl.kernel,
        mesh=mesh,
        out_type=jax.ShapeDtypeStruct((B, D), jnp.float32),
        scratch_types=[
            pltpu.VMEM((n_chunks, CHUNK), jnp.int32),
            pltpu.VMEM((CHUNK, D), jnp.float32),
            pltpu.VMEM((CHUNK, D), jnp.float32),
            pltpu.SemaphoreType.DMA,
            pltpu.SemaphoreType.DMA,
            pltpu.SemaphoreType.DMA,
        ],
    )
    def lookup(idx_hbm, table_hbm, out_hbm, idx_v, buf0, buf1, gsem0, gsem1, osem):
        wid = lax.axis_index("s") * NC + lax.axis_index("c")
        base = wid * b_per_w
        pltpu.sync_copy(idx_hbm.at[wid], idx_v)

        bufs = (buf0, buf1)
        gsems = (gsem0, gsem1)

        pltpu.async_copy(table_hbm.at[idx_v.at[0]], buf0, gsem0)

        def pair(g2, carry):
            for b in range(2):
                g = g2 * 2 + b
                nxt = 1 - b

                @pl.when(g + 1 < n_chunks)
                def _start_next():
                    pltpu.async_copy(
                        table_hbm.at[idx_v.at[g + 1]], bufs[nxt], gsems[nxt]
                    )

                pltpu.make_async_copy(table_hbm.at[idx_v.at[g]], bufs[b], gsems[b]).wait()
                pltpu.sync_copy(bufs[b], out_hbm.at[pl.ds(base + g * CHUNK, CHUNK)])
            return carry

        lax.fori_loop(0, n_chunks // 2, pair, 0)

    return lookup


def kernel(input_ids, embed_weight):
    B = input_ids.shape[0] * input_ids.shape[1]
    D = embed_weight.shape[1]
    info = plsc.get_sparse_core_info()
    NW = info.num_cores * info.num_subcores
    n_chunks = (B // NW) // CHUNK
    idx = input_ids.reshape(NW, n_chunks, CHUNK).astype(jnp.int32)
    out = _make_lookup(B, D)(idx, embed_weight)
    return out.reshape(input_ids.shape[0], input_ids.shape[1], D)
